# Initial kernel scaffold; baseline (speedup 1.0000x reference)
import functools

import jax
import jax.numpy as jnp
from jax import lax
from jax.experimental import pallas as pl
from jax.experimental.pallas import tpu as pltpu
from jax.experimental.pallas import tpu_sc as plsc

NC = 2
NS = 16
CH = 128


def _mlp_bn(nf, Wm, bm, g, be):
    N, D = nf.shape
    H = Wm.shape[1]
    BR = 1000
    NB = N // BR

    def body(nf_ref, wm_ref, bm_ref, g_ref, be_ref, out_ref, acc_ref):
        p = pl.program_id(0)
        i = pl.program_id(1)
        hb = jnp.dot(nf_ref[...], wm_ref[...],
                     preferred_element_type=jnp.float32) + bm_ref[...]
        hb = jnp.where(hb >= 0, hb, 0.1 * hb)

        @pl.when(p == 0)
        def _():
            @pl.when(i == 0)
            def _():
                acc_ref[...] = jnp.zeros_like(acc_ref)
            acc_ref[0:1, :] += jnp.sum(hb, 0, keepdims=True)
            acc_ref[1:2, :] += jnp.sum(hb * hb, 0, keepdims=True)
            out_ref[...] = hb

        @pl.when(p == 1)
        def _():
            mu = acc_ref[0:1, :] / N
            var = acc_ref[1:2, :] / N - mu * mu
            out_ref[...] = ((hb - mu) * lax.rsqrt(var + 1e-5)
                            * g_ref[...] + be_ref[...])

    return pl.pallas_call(
        body,
        grid=(2, NB),
        in_specs=[
            pl.BlockSpec((BR, D), lambda p, i: (i, 0)),
            pl.BlockSpec((D, H), lambda p, i: (0, 0)),
            pl.BlockSpec((1, H), lambda p, i: (0, 0)),
            pl.BlockSpec((1, H), lambda p, i: (0, 0)),
            pl.BlockSpec((1, H), lambda p, i: (0, 0)),
        ],
        out_specs=pl.BlockSpec((BR, H), lambda p, i: (i, 0)),
        out_shape=jax.ShapeDtypeStruct((N, H), jnp.float32),
        scratch_shapes=[pltpu.VMEM((2, H), jnp.float32)],
    )(nf, Wm, bm.reshape(1, H), g.reshape(1, H), be.reshape(1, H))


def _seg_sum(h, src, dst, zrows, zvec, want_deg):
    N, H = h.shape
    E = src.shape[0]
    NW = NC * NS
    total_chunks = E // CH
    base_trips = total_chunks // NW
    extra = total_chunks % NW
    rows_per_sub = N // NS

    deg_off = [s * ((N // NS) // 8 * 8) for s in range(NS)]
    deg_off.append(N)

    mesh = plsc.VectorSubcoreMesh(core_axis_name="c", subcore_axis_name="s",
                                  num_cores=NC, num_subcores=NS)

    out_type = [jax.ShapeDtypeStruct((NC, N, H), jnp.float32)]
    if want_deg:
        out_type.append(jax.ShapeDtypeStruct((NC, N), jnp.float32))

    def body(h_hbm, src_hbm, dst_hbm, zr_hbm, zv_hbm, *rest):
        if want_deg:
            agg_out, deg_out = rest[0], rest[1]
            scratch = rest[2:]
        else:
            agg_out = rest[0]
            scratch = rest[1:]
        sidx, didx, rows, ones_v, sh_agg, sh_deg, sem = scratch

        c = lax.axis_index("c")
        s = lax.axis_index("s")
        wid = c * NS + s

        r0 = s * rows_per_sub
        pltpu.sync_copy(zr_hbm.at[pl.ds(0, rows_per_sub), :],
                        sh_agg.at[pl.ds(r0, rows_per_sub), :])
        if want_deg:
            for si in range(NS):
                @pl.when(s == si)
                def _():
                    d0 = deg_off[si]
                    dn = deg_off[si + 1] - deg_off[si]
                    pltpu.sync_copy(zv_hbm.at[pl.ds(0, dn)],
                                    sh_deg.at[pl.ds(d0, dn)])
            for gi in range(CH // 16):
                ones_v[pl.ds(gi * 16, 16)] = jnp.ones((16,), jnp.float32)
        plsc.subcore_barrier()

        start_chunk = wid * base_trips + jnp.minimum(wid, extra)
        trips = base_trips + jnp.where(wid < extra, 1, 0)

        def chunk(j, carry):
            off = (start_chunk + j) * CH
            pltpu.sync_copy(src_hbm.at[pl.ds(off, CH)], sidx)
            pltpu.sync_copy(dst_hbm.at[pl.ds(off, CH)], didx)
            pltpu.async_copy(h_hbm.at[sidx], rows, sem).wait()
            pltpu.sync_copy(rows, sh_agg.at[didx], add=True)
            if want_deg:
                pltpu.sync_copy(ones_v, sh_deg.at[didx], add=True)
            return carry

        lax.fori_loop(0, trips, chunk, 0)
        plsc.subcore_barrier()

        pltpu.sync_copy(sh_agg.at[pl.ds(r0, rows_per_sub), :],
                        agg_out.at[c, pl.ds(r0, rows_per_sub), :])
        if want_deg:
            for si in range(NS):
                @pl.when(s == si)
                def _():
                    d0 = deg_off[si]
                    dn = deg_off[si + 1] - deg_off[si]
                    pltpu.sync_copy(sh_deg.at[pl.ds(d0, dn)],
                                    deg_out.at[c, pl.ds(d0, dn)])

    scratch_types = [
        pltpu.VMEM((CH,), jnp.int32),
        pltpu.VMEM((CH,), jnp.int32),
        pltpu.VMEM((CH, H), jnp.float32),
        pltpu.VMEM((CH,), jnp.float32),
        pltpu.VMEM_SHARED((N, H), jnp.float32),
        pltpu.VMEM_SHARED((N,), jnp.float32),
        pltpu.SemaphoreType.DMA,
    ]

    fn = pl.kernel(body, out_type=tuple(out_type), mesh=mesh,
                   scratch_types=scratch_types)
    return fn(h, src, dst, zrows, zvec)


def _gather_rows(h, idx):
    N, H = h.shape
    B = idx.shape[0]
    NW = NC * NS
    per_w = B // NW

    mesh = plsc.VectorSubcoreMesh(core_axis_name="c", subcore_axis_name="s",
                                  num_cores=NC, num_subcores=NS)

    def body(h_hbm, idx_hbm, out_hbm, idxv, rowsv, sem):
        c = lax.axis_index("c")
        s = lax.axis_index("s")
        wid = c * NS + s
        off = wid * per_w
        pltpu.sync_copy(idx_hbm.at[pl.ds(off, per_w)], idxv)
        pltpu.async_copy(h_hbm.at[idxv], rowsv, sem).wait()
        pltpu.sync_copy(rowsv, out_hbm.at[pl.ds(off, per_w), :])

    fn = pl.kernel(
        body,
        out_type=jax.ShapeDtypeStruct((B, H), jnp.float32),
        mesh=mesh,
        scratch_types=[
            pltpu.VMEM((per_w,), jnp.int32),
            pltpu.VMEM((per_w, H), jnp.float32),
            pltpu.SemaphoreType.DMA,
        ])
    return fn(h, idx)


def _sage_layer(h, aggp, degp, Ws, Wn, bs):
    N, H = h.shape
    BR = 1000
    NB = N // BR

    def body(h_ref, aggp_ref, degp_ref, ws_ref, wn_ref, bs_ref, out_ref):
        d = jnp.maximum(degp_ref[0] + degp_ref[1], 1.0)
        agg = (aggp_ref[0] + aggp_ref[1]) / d
        o = (jnp.dot(h_ref[...], ws_ref[...], preferred_element_type=jnp.float32)
             + jnp.dot(agg, wn_ref[...], preferred_element_type=jnp.float32)
             + bs_ref[...])
        out_ref[...] = jnp.where(o >= 0, o, 0.1 * o)

    return pl.pallas_call(
        body,
        grid=(NB,),
        in_specs=[
            pl.BlockSpec((BR, H), lambda i: (i, 0)),
            pl.BlockSpec((NC, BR, H), lambda i: (0, i, 0)),
            pl.BlockSpec((NC, BR, 1), lambda i: (0, i, 0)),
            pl.BlockSpec((H, H), lambda i: (0, 0)),
            pl.BlockSpec((H, H), lambda i: (0, 0)),
            pl.BlockSpec((1, H), lambda i: (0, 0)),
        ],
        out_specs=pl.BlockSpec((BR, H), lambda i: (i, 0)),
        out_shape=jax.ShapeDtypeStruct((N, H), jnp.float32),
    )(h, aggp, degp.reshape(NC, N, 1), Ws, Wn, bs.reshape(1, H))


def _loss(hs, x, Wp, bp):
    B, H = hs.shape
    O = Wp.shape[1]

    def body(hs_ref, x_ref, wp_ref, bp_ref, out_ref):
        pred = (jnp.dot(hs_ref[...], wp_ref[...],
                        preferred_element_type=jnp.float32)
                + bp_ref[...] - x_ref[...])
        out_ref[0, 0] = jnp.sum(pred * pred) / (B * O)

    return pl.pallas_call(
        body,
        out_shape=jax.ShapeDtypeStruct((1, 1), jnp.float32),
    )(hs, x, Wp, bp.reshape(1, O))


def _panel_impl(nf, edge_index, tgt, x, Wm, bm, g, be,
                Ws0, Wn0, bs0, Ws1, Wn1, bs1, Wp, bp, zrows, zvec):
    src = edge_index[0]
    dst = edge_index[1]

    h = _mlp_bn(nf, Wm, bm, g, be)
    aggp, degp = _seg_sum(h, src, dst, zrows, zvec, want_deg=True)
    h1 = _sage_layer(h, aggp, degp, Ws0, Wn0, bs0)
    (agg1p,) = _seg_sum(h1, src, dst, zrows, zvec, want_deg=False)
    h2 = _sage_layer(h1, agg1p, degp, Ws1, Wn1, bs1)
    hs = _gather_rows(h2, tgt)
    return _loss(hs, x, Wp, bp)[0, 0]


def kernel(node_feat1, node_feat2, x1, x2, edge_index1, edge_index2,
           tgt_id1, tgt_id2,
           Wm1, bm1, g1, be1, Ws10, Wn10, bs10, Ws11, Wn11, bs11, Wp1, bp1,
           Wm2, bm2, g2, be2, Ws20, Wn20, bs20, Ws21, Wn21, bs21, Wp2, bp2):
    N, D = node_feat1.shape
    H = Wm1.shape[1]
    zrows = jnp.zeros((N // NS, H), jnp.float32)
    zvec = jnp.zeros((N,), jnp.float32)
    l1 = _panel_impl(node_feat1, edge_index1, tgt_id1, x1, Wm1, bm1, g1, be1,
                     Ws10, Wn10, bs10, Ws11, Wn11, bs11, Wp1, bp1, zrows, zvec)
    l2 = _panel_impl(node_feat2, edge_index2, tgt_id2, x2, Wm2, bm2, g2, be2,
                     Ws20, Wn20, bs20, Ws21, Wn21, bs21, Wp2, bp2, zrows, zvec)
    return jnp.stack([l1, l2])


# speedup vs baseline: 4.3338x; 4.3338x over previous
import functools

import jax
import jax.numpy as jnp
from jax import lax
from jax.experimental import pallas as pl
from jax.experimental.pallas import tpu as pltpu
from jax.experimental.pallas import tpu_sc as plsc

NC = 2
NS = 16
CH = 128


def _mlp_bn(nf, Wm, bm, g, be):
    N, D = nf.shape
    H = Wm.shape[1]
    BR = 1000
    NB = N // BR

    def body(nf_ref, wm_ref, bm_ref, g_ref, be_ref, out_ref, acc_ref):
        p = pl.program_id(0)
        i = pl.program_id(1)
        hb = jnp.dot(nf_ref[...], wm_ref[...],
                     preferred_element_type=jnp.float32) + bm_ref[...]
        hb = jnp.where(hb >= 0, hb, 0.1 * hb)

        @pl.when(p == 0)
        def _():
            @pl.when(i == 0)
            def _():
                acc_ref[...] = jnp.zeros_like(acc_ref)
            acc_ref[0:1, :] += jnp.sum(hb, 0, keepdims=True)
            acc_ref[1:2, :] += jnp.sum(hb * hb, 0, keepdims=True)
            out_ref[...] = hb

        @pl.when(p == 1)
        def _():
            mu = acc_ref[0:1, :] / N
            var = acc_ref[1:2, :] / N - mu * mu
            out_ref[...] = ((hb - mu) * lax.rsqrt(var + 1e-5)
                            * g_ref[...] + be_ref[...])

    return pl.pallas_call(
        body,
        grid=(2, NB),
        in_specs=[
            pl.BlockSpec((BR, D), lambda p, i: (i, 0)),
            pl.BlockSpec((D, H), lambda p, i: (0, 0)),
            pl.BlockSpec((1, H), lambda p, i: (0, 0)),
            pl.BlockSpec((1, H), lambda p, i: (0, 0)),
            pl.BlockSpec((1, H), lambda p, i: (0, 0)),
        ],
        out_specs=pl.BlockSpec((BR, H), lambda p, i: (i, 0)),
        out_shape=jax.ShapeDtypeStruct((N, H), jnp.float32),
        scratch_shapes=[pltpu.VMEM((2, H), jnp.float32)],
    )(nf, Wm, bm.reshape(1, H), g.reshape(1, H), be.reshape(1, H))


def _seg_sum(h, src, dst, zrows, want_deg):
    N, H = h.shape
    E = src.shape[0]
    NW = NC * NS
    total_chunks = E // CH
    base_trips = total_chunks // NW
    extra = total_chunks % NW
    row_off = [s * ((N // NS) // 8 * 8) for s in range(NS)]
    row_off.append(N)
    max_rows = max(row_off[s + 1] - row_off[s] for s in range(NS))

    mesh = plsc.VectorSubcoreMesh(core_axis_name="c", subcore_axis_name="s",
                                  num_cores=NC, num_subcores=NS)

    out_type = [jax.ShapeDtypeStruct((NC, N, H), jnp.float32)]
    if want_deg:
        out_type.append(jax.ShapeDtypeStruct((NC * N,), jnp.float32))

    def body(h_hbm, src_hbm, dst_hbm, zr_hbm, *rest):
        if want_deg:
            agg_out, deg_out = rest[0], rest[1]
            scratch = rest[2:]
        else:
            agg_out = rest[0]
            scratch = rest[1:]
        sidx, didx, rows, ones_v, degv, sh_agg, sh_deg, sem = scratch

        c = lax.axis_index("c")
        s = lax.axis_index("s")
        wid = c * NS + s

        if want_deg:
            for gi in range(max_rows // 16):
                degv[pl.ds(gi * 16, 16)] = jnp.zeros((16,), jnp.float32)
            for gi in range(CH // 16):
                ones_v[pl.ds(gi * 16, 16)] = jnp.ones((16,), jnp.float32)
        for si in range(NS):
            @pl.when(s == si)
            def _():
                d0 = row_off[si]
                dn = row_off[si + 1] - row_off[si]
                pltpu.sync_copy(zr_hbm.at[pl.ds(0, dn), :],
                                sh_agg.at[pl.ds(d0, dn), :])
                if want_deg:
                    pltpu.sync_copy(degv.at[pl.ds(0, dn)],
                                    sh_deg.at[pl.ds(d0, dn)])
        plsc.subcore_barrier()

        start_chunk = wid * base_trips + jnp.minimum(wid, extra)
        trips = base_trips + jnp.where(wid < extra, 1, 0)

        def chunk(j, carry):
            off = (start_chunk + j) * CH
            pltpu.sync_copy(src_hbm.at[pl.ds(off, CH)], sidx)
            pltpu.sync_copy(dst_hbm.at[pl.ds(off, CH)], didx)
            pltpu.async_copy(h_hbm.at[sidx], rows, sem).wait()
            pltpu.sync_copy(rows, sh_agg.at[didx], add=True)
            if want_deg:
                pltpu.sync_copy(ones_v, sh_deg.at[didx], add=True)
            return carry

        lax.fori_loop(0, trips, chunk, 0)
        plsc.subcore_barrier()

        for si in range(NS):
            @pl.when(s == si)
            def _():
                d0 = row_off[si]
                dn = row_off[si + 1] - row_off[si]
                pltpu.sync_copy(sh_agg.at[pl.ds(d0, dn), :],
                                agg_out.at[c, pl.ds(d0, dn), :])
                if want_deg:
                    pltpu.sync_copy(sh_deg.at[pl.ds(d0, dn)],
                                    degv.at[pl.ds(0, dn)])
                    pltpu.sync_copy(degv.at[pl.ds(0, dn)],
                                    deg_out.at[pl.ds(c * N + d0, dn)])

    scratch_types = [
        pltpu.VMEM((CH,), jnp.int32),
        pltpu.VMEM((CH,), jnp.int32),
        pltpu.VMEM((CH, H), jnp.float32),
        pltpu.VMEM((CH,), jnp.float32),
        pltpu.VMEM((max_rows,), jnp.float32),
        pltpu.VMEM_SHARED((N, H), jnp.float32),
        pltpu.VMEM_SHARED((N,), jnp.float32),
        pltpu.SemaphoreType.DMA,
    ]

    fn = pl.kernel(body, out_type=tuple(out_type), mesh=mesh,
                   scratch_types=scratch_types)
    return fn(h, src, dst, zrows)


def _gather_rows(h, idx):
    N, H = h.shape
    B = idx.shape[0]
    NW = NC * NS
    per_w = B // NW

    mesh = plsc.VectorSubcoreMesh(core_axis_name="c", subcore_axis_name="s",
                                  num_cores=NC, num_subcores=NS)

    def body(h_hbm, idx_hbm, out_hbm, idxv, rowsv, sem):
        c = lax.axis_index("c")
        s = lax.axis_index("s")
        wid = c * NS + s
        off = wid * per_w
        pltpu.sync_copy(idx_hbm.at[pl.ds(off, per_w)], idxv)
        pltpu.async_copy(h_hbm.at[idxv], rowsv, sem).wait()
        pltpu.sync_copy(rowsv, out_hbm.at[pl.ds(off, per_w), :])

    fn = pl.kernel(
        body,
        out_type=jax.ShapeDtypeStruct((B, H), jnp.float32),
        mesh=mesh,
        scratch_types=[
            pltpu.VMEM((per_w,), jnp.int32),
            pltpu.VMEM((per_w, H), jnp.float32),
            pltpu.SemaphoreType.DMA,
        ])
    return fn(h, idx)


def _sage_layer(h, aggp, degp, Ws, Wn, bs):
    N, H = h.shape
    BR = 1000
    NB = N // BR

    def body(h_ref, aggp_ref, degp_ref, ws_ref, wn_ref, bs_ref, out_ref):
        d = jnp.maximum(degp_ref[0] + degp_ref[1], 1.0)
        agg = (aggp_ref[0] + aggp_ref[1]) / d
        o = (jnp.dot(h_ref[...], ws_ref[...], preferred_element_type=jnp.float32)
             + jnp.dot(agg, wn_ref[...], preferred_element_type=jnp.float32)
             + bs_ref[...])
        out_ref[...] = jnp.where(o >= 0, o, 0.1 * o)

    return pl.pallas_call(
        body,
        grid=(NB,),
        in_specs=[
            pl.BlockSpec((BR, H), lambda i: (i, 0)),
            pl.BlockSpec((NC, BR, H), lambda i: (0, i, 0)),
            pl.BlockSpec((NC, BR, 1), lambda i: (0, i, 0)),
            pl.BlockSpec((H, H), lambda i: (0, 0)),
            pl.BlockSpec((H, H), lambda i: (0, 0)),
            pl.BlockSpec((1, H), lambda i: (0, 0)),
        ],
        out_specs=pl.BlockSpec((BR, H), lambda i: (i, 0)),
        out_shape=jax.ShapeDtypeStruct((N, H), jnp.float32),
    )(h, aggp, degp.reshape(NC, N, 1), Ws, Wn, bs.reshape(1, H))


def _loss(hs, x, Wp, bp):
    B, H = hs.shape
    O = Wp.shape[1]

    def body(hs_ref, x_ref, wp_ref, bp_ref, out_ref):
        pred = (jnp.dot(hs_ref[...], wp_ref[...],
                        preferred_element_type=jnp.float32)
                + bp_ref[...] - x_ref[...])
        out_ref[...] = jnp.sum(pred * pred).reshape(1, 1) / (B * O)

    return pl.pallas_call(
        body,
        out_shape=jax.ShapeDtypeStruct((1, 1), jnp.float32),
    )(hs, x, Wp, bp.reshape(1, O))


def _panel_impl(nf, edge_index, tgt, x, Wm, bm, g, be,
                Ws0, Wn0, bs0, Ws1, Wn1, bs1, Wp, bp, zrows):
    src = edge_index[0]
    dst = edge_index[1]

    h = _mlp_bn(nf, Wm, bm, g, be)
    aggp, degp = _seg_sum(h, src, dst, zrows, want_deg=True)
    degp = degp.reshape(NC, h.shape[0])
    h1 = _sage_layer(h, aggp, degp, Ws0, Wn0, bs0)
    (agg1p,) = _seg_sum(h1, src, dst, zrows, want_deg=False)
    h2 = _sage_layer(h1, agg1p, degp, Ws1, Wn1, bs1)
    hs = _gather_rows(h2, tgt)
    return _loss(hs, x, Wp, bp)[0, 0]


def kernel(node_feat1, node_feat2, x1, x2, edge_index1, edge_index2,
           tgt_id1, tgt_id2,
           Wm1, bm1, g1, be1, Ws10, Wn10, bs10, Ws11, Wn11, bs11, Wp1, bp1,
           Wm2, bm2, g2, be2, Ws20, Wn20, bs20, Ws21, Wn21, bs21, Wp2, bp2):
    N, D = node_feat1.shape
    H = Wm1.shape[1]
    max_rows = N - (NS - 1) * ((N // NS) // 8 * 8)
    zrows = jnp.zeros((max_rows, H), jnp.float32)
    l1 = _panel_impl(node_feat1, edge_index1, tgt_id1, x1, Wm1, bm1, g1, be1,
                     Ws10, Wn10, bs10, Ws11, Wn11, bs11, Wp1, bp1, zrows)
    l2 = _panel_impl(node_feat2, edge_index2, tgt_id2, x2, Wm2, bm2, g2, be2,
                     Ws20, Wn20, bs20, Ws21, Wn21, bs21, Wp2, bp2, zrows)
    return jnp.stack([l1, l2])


# speedup vs baseline: 8.0956x; 1.8680x over previous
import functools

import jax
import jax.numpy as jnp
from jax import lax
from jax.experimental import pallas as pl
from jax.experimental.pallas import tpu as pltpu
from jax.experimental.pallas import tpu_sc as plsc

NC = 2
NS = 16
CH = 128


def _mlp_bn(nf, Wm, bm, g, be):
    N, D = nf.shape
    H = Wm.shape[1]
    BR = 1000
    NB = N // BR

    def body(nf_ref, wm_ref, bm_ref, g_ref, be_ref, out_ref, acc_ref):
        p = pl.program_id(0)
        i = pl.program_id(1)
        hb = jnp.dot(nf_ref[...], wm_ref[...],
                     preferred_element_type=jnp.float32) + bm_ref[...]
        hb = jnp.where(hb >= 0, hb, 0.1 * hb)

        @pl.when(p == 0)
        def _():
            @pl.when(i == 0)
            def _():
                acc_ref[...] = jnp.zeros_like(acc_ref)
            acc_ref[0:1, :] += jnp.sum(hb, 0, keepdims=True)
            acc_ref[1:2, :] += jnp.sum(hb * hb, 0, keepdims=True)
            out_ref[...] = hb

        @pl.when(p == 1)
        def _():
            mu = acc_ref[0:1, :] / N
            var = acc_ref[1:2, :] / N - mu * mu
            out_ref[...] = ((hb - mu) * lax.rsqrt(var + 1e-5)
                            * g_ref[...] + be_ref[...])

    return pl.pallas_call(
        body,
        grid=(2, NB),
        in_specs=[
            pl.BlockSpec((BR, D), lambda p, i: (i, 0)),
            pl.BlockSpec((D, H), lambda p, i: (0, 0)),
            pl.BlockSpec((1, H), lambda p, i: (0, 0)),
            pl.BlockSpec((1, H), lambda p, i: (0, 0)),
            pl.BlockSpec((1, H), lambda p, i: (0, 0)),
        ],
        out_specs=pl.BlockSpec((BR, H), lambda p, i: (i, 0)),
        out_shape=jax.ShapeDtypeStruct((N, H), jnp.float32),
        scratch_shapes=[pltpu.VMEM((2, H), jnp.float32)],
    )(nf, Wm, bm.reshape(1, H), g.reshape(1, H), be.reshape(1, H))


def _seg_sum(h, src, dst, zrows, want_deg):
    N, H = h.shape
    E = src.shape[0]
    NW = NC * NS
    EW = E // NW
    SCH = 80
    nch = EW // SCH
    assert E == NW * EW and EW == nch * SCH and nch % 2 == 1
    row_off = [s * ((N // NS) // 8 * 8) for s in range(NS)]
    row_off.append(N)
    max_rows = max(row_off[s + 1] - row_off[s] for s in range(NS))

    mesh = plsc.VectorSubcoreMesh(core_axis_name="c", subcore_axis_name="s",
                                  num_cores=NC, num_subcores=NS)

    out_type = [jax.ShapeDtypeStruct((NC, N, H), jnp.float32)]
    if want_deg:
        out_type.append(jax.ShapeDtypeStruct((NC * N,), jnp.float32))

    def body(h_hbm, src_hbm, dst_hbm, zr_hbm, *rest):
        if want_deg:
            agg_out, deg_out = rest[0], rest[1]
            scratch = rest[2:]
        else:
            agg_out = rest[0]
            scratch = rest[1:]
        (sall, dall, d0i, d1i, rows0, rows1, ones_v, degv,
         sh_agg, sh_deg, gsem0, gsem1, dsem) = scratch

        c = lax.axis_index("c")
        s = lax.axis_index("s")
        wid = c * NS + s
        base = wid * EW

        pltpu.sync_copy(src_hbm.at[pl.ds(base, EW)], sall)
        pltpu.sync_copy(dst_hbm.at[pl.ds(base, EW)], dall)

        if want_deg:
            for gi in range(max_rows // 16):
                degv[pl.ds(gi * 16, 16)] = jnp.zeros((16,), jnp.float32)
            for gi in range(SCH // 16):
                ones_v[pl.ds(gi * 16, 16)] = jnp.ones((16,), jnp.float32)
        for si in range(NS):
            @pl.when(s == si)
            def _():
                d0 = row_off[si]
                dn = row_off[si + 1] - row_off[si]
                pltpu.sync_copy(zr_hbm.at[pl.ds(0, dn), :],
                                sh_agg.at[pl.ds(d0, dn), :])
                if want_deg:
                    pltpu.sync_copy(degv.at[pl.ds(0, dn)],
                                    sh_deg.at[pl.ds(d0, dn)])
        plsc.subcore_barrier()

        def copy_didx(dref, ch):
            for gi in range(SCH // 16):
                dref[pl.ds(gi * 16, 16)] = dall[pl.ds(ch * SCH + gi * 16, 16)]

        def gstart(rref, ch, sem):
            pltpu.async_copy(h_hbm.at[sall.at[pl.ds(ch * SCH, SCH)]],
                             rref, sem)

        def gwait(rref, sem):
            pltpu.make_async_copy(h_hbm.at[pl.ds(0, SCH), :], rref, sem).wait()

        def scat(rref, dref):
            if want_deg:
                ddesc = pltpu.async_copy(ones_v, sh_deg.at[dref], dsem,
                                         add=True)
                pltpu.sync_copy(rref, sh_agg.at[dref], add=True)
                ddesc.wait()
            else:
                pltpu.sync_copy(rref, sh_agg.at[dref], add=True)

        copy_didx(d0i, 0)
        gstart(rows0, 0, gsem0)

        def pair(p, carry):
            c0 = 2 * p
            copy_didx(d1i, c0 + 1)
            gstart(rows1, c0 + 1, gsem1)
            gwait(rows0, gsem0)
            scat(rows0, d0i)
            copy_didx(d0i, c0 + 2)
            gstart(rows0, c0 + 2, gsem0)
            gwait(rows1, gsem1)
            scat(rows1, d1i)
            return carry

        lax.fori_loop(0, (nch - 1) // 2, pair, 0)
        gwait(rows0, gsem0)
        scat(rows0, d0i)
        plsc.subcore_barrier()

        for si in range(NS):
            @pl.when(s == si)
            def _():
                d0 = row_off[si]
                dn = row_off[si + 1] - row_off[si]
                pltpu.sync_copy(sh_agg.at[pl.ds(d0, dn), :],
                                agg_out.at[c, pl.ds(d0, dn), :])
                if want_deg:
                    pltpu.sync_copy(sh_deg.at[pl.ds(d0, dn)],
                                    degv.at[pl.ds(0, dn)])
                    pltpu.sync_copy(degv.at[pl.ds(0, dn)],
                                    deg_out.at[pl.ds(c * N + d0, dn)])

    scratch_types = [
        pltpu.VMEM((EW,), jnp.int32),
        pltpu.VMEM((EW,), jnp.int32),
        pltpu.VMEM((SCH,), jnp.int32),
        pltpu.VMEM((SCH,), jnp.int32),
        pltpu.VMEM((SCH, H), jnp.float32),
        pltpu.VMEM((SCH, H), jnp.float32),
        pltpu.VMEM((SCH,), jnp.float32),
        pltpu.VMEM((max_rows,), jnp.float32),
        pltpu.VMEM_SHARED((N, H), jnp.float32),
        pltpu.VMEM_SHARED((N,), jnp.float32),
        pltpu.SemaphoreType.DMA,
        pltpu.SemaphoreType.DMA,
        pltpu.SemaphoreType.DMA,
    ]

    fn = pl.kernel(body, out_type=tuple(out_type), mesh=mesh,
                   scratch_types=scratch_types)
    return fn(h, src, dst, zrows)


def _gather_rows(h, idx):
    N, H = h.shape
    B = idx.shape[0]
    NW = NC * NS
    per_w = B // NW

    mesh = plsc.VectorSubcoreMesh(core_axis_name="c", subcore_axis_name="s",
                                  num_cores=NC, num_subcores=NS)

    def body(h_hbm, idx_hbm, out_hbm, idxv, rowsv, sem):
        c = lax.axis_index("c")
        s = lax.axis_index("s")
        wid = c * NS + s
        off = wid * per_w
        pltpu.sync_copy(idx_hbm.at[pl.ds(off, per_w)], idxv)
        pltpu.async_copy(h_hbm.at[idxv], rowsv, sem).wait()
        pltpu.sync_copy(rowsv, out_hbm.at[pl.ds(off, per_w), :])

    fn = pl.kernel(
        body,
        out_type=jax.ShapeDtypeStruct((B, H), jnp.float32),
        mesh=mesh,
        scratch_types=[
            pltpu.VMEM((per_w,), jnp.int32),
            pltpu.VMEM((per_w, H), jnp.float32),
            pltpu.SemaphoreType.DMA,
        ])
    return fn(h, idx)


def _sage_layer(h, aggp, degp, Ws, Wn, bs):
    N, H = h.shape
    BR = 1000
    NB = N // BR

    def body(h_ref, aggp_ref, degp_ref, ws_ref, wn_ref, bs_ref, out_ref):
        d = jnp.maximum(degp_ref[0] + degp_ref[1], 1.0)
        agg = (aggp_ref[0] + aggp_ref[1]) / d
        o = (jnp.dot(h_ref[...], ws_ref[...], preferred_element_type=jnp.float32)
             + jnp.dot(agg, wn_ref[...], preferred_element_type=jnp.float32)
             + bs_ref[...])
        out_ref[...] = jnp.where(o >= 0, o, 0.1 * o)

    return pl.pallas_call(
        body,
        grid=(NB,),
        in_specs=[
            pl.BlockSpec((BR, H), lambda i: (i, 0)),
            pl.BlockSpec((NC, BR, H), lambda i: (0, i, 0)),
            pl.BlockSpec((NC, BR, 1), lambda i: (0, i, 0)),
            pl.BlockSpec((H, H), lambda i: (0, 0)),
            pl.BlockSpec((H, H), lambda i: (0, 0)),
            pl.BlockSpec((1, H), lambda i: (0, 0)),
        ],
        out_specs=pl.BlockSpec((BR, H), lambda i: (i, 0)),
        out_shape=jax.ShapeDtypeStruct((N, H), jnp.float32),
    )(h, aggp, degp.reshape(NC, N, 1), Ws, Wn, bs.reshape(1, H))


def _loss(hs, x, Wp, bp):
    B, H = hs.shape
    O = Wp.shape[1]

    def body(hs_ref, x_ref, wp_ref, bp_ref, out_ref):
        pred = (jnp.dot(hs_ref[...], wp_ref[...],
                        preferred_element_type=jnp.float32)
                + bp_ref[...] - x_ref[...])
        out_ref[...] = jnp.sum(pred * pred).reshape(1, 1) / (B * O)

    return pl.pallas_call(
        body,
        out_shape=jax.ShapeDtypeStruct((1, 1), jnp.float32),
    )(hs, x, Wp, bp.reshape(1, O))


def _panel_impl(nf, edge_index, tgt, x, Wm, bm, g, be,
                Ws0, Wn0, bs0, Ws1, Wn1, bs1, Wp, bp, zrows):
    src = edge_index[0]
    dst = edge_index[1]

    h = _mlp_bn(nf, Wm, bm, g, be)
    aggp, degp = _seg_sum(h, src, dst, zrows, want_deg=True)
    degp = degp.reshape(NC, h.shape[0])
    h1 = _sage_layer(h, aggp, degp, Ws0, Wn0, bs0)
    (agg1p,) = _seg_sum(h1, src, dst, zrows, want_deg=False)
    h2 = _sage_layer(h1, agg1p, degp, Ws1, Wn1, bs1)
    hs = _gather_rows(h2, tgt)
    return _loss(hs, x, Wp, bp)[0, 0]


def kernel(node_feat1, node_feat2, x1, x2, edge_index1, edge_index2,
           tgt_id1, tgt_id2,
           Wm1, bm1, g1, be1, Ws10, Wn10, bs10, Ws11, Wn11, bs11, Wp1, bp1,
           Wm2, bm2, g2, be2, Ws20, Wn20, bs20, Ws21, Wn21, bs21, Wp2, bp2):
    N, D = node_feat1.shape
    H = Wm1.shape[1]
    max_rows = N - (NS - 1) * ((N // NS) // 8 * 8)
    zrows = jnp.zeros((max_rows, H), jnp.float32)
    l1 = _panel_impl(node_feat1, edge_index1, tgt_id1, x1, Wm1, bm1, g1, be1,
                     Ws10, Wn10, bs10, Ws11, Wn11, bs11, Wp1, bp1, zrows)
    l2 = _panel_impl(node_feat2, edge_index2, tgt_id2, x2, Wm2, bm2, g2, be2,
                     Ws20, Wn20, bs20, Ws21, Wn21, bs21, Wp2, bp2, zrows)
    return jnp.stack([l1, l2])


# speedup vs baseline: 8.1095x; 1.0017x over previous
import jax
import jax.numpy as jnp
from jax import lax
from jax.experimental import pallas as pl
from jax.experimental.pallas import tpu as pltpu
from jax.experimental.pallas import tpu_sc as plsc

NC = 2
NS = 16


def _mlp_bn(nf, Wm, bm, g, be):
    N, D = nf.shape
    H = Wm.shape[1]
    BR = 1000
    NB = N // BR

    def body(nf_ref, wm_ref, bm_ref, g_ref, be_ref, out_ref, acc_ref):
        p = pl.program_id(0)
        i = pl.program_id(1)
        hb = jnp.dot(nf_ref[...], wm_ref[...],
                     preferred_element_type=jnp.float32) + bm_ref[...]
        hb = jnp.where(hb >= 0, hb, 0.1 * hb)

        @pl.when(p == 0)
        def _():
            @pl.when(i == 0)
            def _():
                acc_ref[...] = jnp.zeros_like(acc_ref)
            acc_ref[0:1, :] += jnp.sum(hb, 0, keepdims=True)
            acc_ref[1:2, :] += jnp.sum(hb * hb, 0, keepdims=True)
            out_ref[...] = hb

        @pl.when(p == 1)
        def _():
            mu = acc_ref[0:1, :] / N
            var = acc_ref[1:2, :] / N - mu * mu
            out_ref[...] = ((hb - mu) * lax.rsqrt(var + 1e-5)
                            * g_ref[...] + be_ref[...])

    return pl.pallas_call(
        body,
        grid=(2, NB),
        in_specs=[
            pl.BlockSpec((BR, D), lambda p, i: (i, 0)),
            pl.BlockSpec((D, H), lambda p, i: (0, 0)),
            pl.BlockSpec((1, H), lambda p, i: (0, 0)),
            pl.BlockSpec((1, H), lambda p, i: (0, 0)),
            pl.BlockSpec((1, H), lambda p, i: (0, 0)),
        ],
        out_specs=pl.BlockSpec((BR, H), lambda p, i: (i, 0)),
        out_shape=jax.ShapeDtypeStruct((N, H), jnp.float32),
        scratch_shapes=[pltpu.VMEM((2, H), jnp.float32)],
    )(nf, Wm, bm.reshape(1, H), g.reshape(1, H), be.reshape(1, H))


def _seg_sum(h, src, dst, zrows, want_deg):
    N, H = h.shape
    E = src.shape[0]
    NW = NC * NS
    EW = E // NW
    SCH = 80
    nch = EW // SCH
    assert E == NW * EW and EW == nch * SCH and nch % 2 == 1
    row_off = [s * ((N // NS) // 8 * 8) for s in range(NS)]
    row_off.append(N)
    max_rows = max(row_off[s + 1] - row_off[s] for s in range(NS))

    mesh = plsc.VectorSubcoreMesh(core_axis_name="c", subcore_axis_name="s",
                                  num_cores=NC, num_subcores=NS)

    out_type = [jax.ShapeDtypeStruct((NC, N, H), jnp.float32)]
    if want_deg:
        out_type.append(jax.ShapeDtypeStruct((NC * N,), jnp.float32))

    def body(h_hbm, src_hbm, dst_hbm, zr_hbm, *rest):
        if want_deg:
            agg_out, deg_out = rest[0], rest[1]
            scratch = rest[2:]
        else:
            agg_out = rest[0]
            scratch = rest[1:]
        (sall, dall, d0i, d1i, rows0, rows1, ones_v, degv,
         sh_agg, sh_deg, gsem0, gsem1, dsem) = scratch

        c = lax.axis_index("c")
        s = lax.axis_index("s")
        wid = c * NS + s
        base = wid * EW

        pltpu.sync_copy(src_hbm.at[pl.ds(base, EW)], sall)
        pltpu.sync_copy(dst_hbm.at[pl.ds(base, EW)], dall)

        if want_deg:
            for gi in range(max_rows // 16):
                degv[pl.ds(gi * 16, 16)] = jnp.zeros((16,), jnp.float32)
            for gi in range(SCH // 16):
                ones_v[pl.ds(gi * 16, 16)] = jnp.ones((16,), jnp.float32)
        for si in range(NS):
            @pl.when(s == si)
            def _():
                d0 = row_off[si]
                dn = row_off[si + 1] - row_off[si]
                pltpu.sync_copy(zr_hbm.at[pl.ds(0, dn), :],
                                sh_agg.at[pl.ds(d0, dn), :])
                if want_deg:
                    pltpu.sync_copy(degv.at[pl.ds(0, dn)],
                                    sh_deg.at[pl.ds(d0, dn)])
        plsc.subcore_barrier()

        def copy_didx(dref, ch):
            for gi in range(SCH // 16):
                dref[pl.ds(gi * 16, 16)] = dall[pl.ds(ch * SCH + gi * 16, 16)]

        def gstart(rref, ch, sem):
            pltpu.async_copy(h_hbm.at[sall.at[pl.ds(ch * SCH, SCH)]],
                             rref, sem)

        def gwait(rref, sem):
            pltpu.make_async_copy(h_hbm.at[pl.ds(0, SCH), :], rref, sem).wait()

        def scat(rref, dref):
            if want_deg:
                ddesc = pltpu.async_copy(ones_v, sh_deg.at[dref], dsem,
                                         add=True)
                pltpu.sync_copy(rref, sh_agg.at[dref], add=True)
                ddesc.wait()
            else:
                pltpu.sync_copy(rref, sh_agg.at[dref], add=True)

        copy_didx(d0i, 0)
        gstart(rows0, 0, gsem0)

        def pair(p, carry):
            c0 = 2 * p
            copy_didx(d1i, c0 + 1)
            gstart(rows1, c0 + 1, gsem1)
            gwait(rows0, gsem0)
            scat(rows0, d0i)
            copy_didx(d0i, c0 + 2)
            gstart(rows0, c0 + 2, gsem0)
            gwait(rows1, gsem1)
            scat(rows1, d1i)
            return carry

        lax.fori_loop(0, (nch - 1) // 2, pair, 0)
        gwait(rows0, gsem0)
        scat(rows0, d0i)
        plsc.subcore_barrier()

        for si in range(NS):
            @pl.when(s == si)
            def _():
                d0 = row_off[si]
                dn = row_off[si + 1] - row_off[si]
                pltpu.sync_copy(sh_agg.at[pl.ds(d0, dn), :],
                                agg_out.at[c, pl.ds(d0, dn), :])
                if want_deg:
                    pltpu.sync_copy(sh_deg.at[pl.ds(d0, dn)],
                                    degv.at[pl.ds(0, dn)])
                    pltpu.sync_copy(degv.at[pl.ds(0, dn)],
                                    deg_out.at[pl.ds(c * N + d0, dn)])

    scratch_types = [
        pltpu.VMEM((EW,), jnp.int32),
        pltpu.VMEM((EW,), jnp.int32),
        pltpu.VMEM((SCH,), jnp.int32),
        pltpu.VMEM((SCH,), jnp.int32),
        pltpu.VMEM((SCH, H), jnp.float32),
        pltpu.VMEM((SCH, H), jnp.float32),
        pltpu.VMEM((SCH,), jnp.float32),
        pltpu.VMEM((max_rows,), jnp.float32),
        pltpu.VMEM_SHARED((N, H), jnp.float32),
        pltpu.VMEM_SHARED((N,), jnp.float32),
        pltpu.SemaphoreType.DMA,
        pltpu.SemaphoreType.DMA,
        pltpu.SemaphoreType.DMA,
    ]

    fn = pl.kernel(body, out_type=tuple(out_type), mesh=mesh,
                   scratch_types=scratch_types)
    return fn(h, src, dst, zrows)


def _gather_rows(h, idx):
    N, H = h.shape
    B = idx.shape[0]
    NW = NC * NS
    per_w = B // NW

    mesh = plsc.VectorSubcoreMesh(core_axis_name="c", subcore_axis_name="s",
                                  num_cores=NC, num_subcores=NS)

    def body(h_hbm, idx_hbm, out_hbm, idxv, rowsv, sem):
        c = lax.axis_index("c")
        s = lax.axis_index("s")
        wid = c * NS + s
        off = wid * per_w
        pltpu.sync_copy(idx_hbm.at[pl.ds(off, per_w)], idxv)
        pltpu.async_copy(h_hbm.at[idxv], rowsv, sem).wait()
        pltpu.sync_copy(rowsv, out_hbm.at[pl.ds(off, per_w), :])

    fn = pl.kernel(
        body,
        out_type=jax.ShapeDtypeStruct((B, H), jnp.float32),
        mesh=mesh,
        scratch_types=[
            pltpu.VMEM((per_w,), jnp.int32),
            pltpu.VMEM((per_w, H), jnp.float32),
            pltpu.SemaphoreType.DMA,
        ])
    return fn(h, idx)


def _sage_layer(h, aggp, degp, Ws, Wn, bs):
    N, H = h.shape
    BR = 1000
    NB = N // BR

    def body(h_ref, aggp_ref, degp_ref, ws_ref, wn_ref, bs_ref, out_ref):
        d = jnp.maximum(degp_ref[0] + degp_ref[1], 1.0)
        agg = (aggp_ref[0] + aggp_ref[1]) / d
        o = (jnp.dot(h_ref[...], ws_ref[...], preferred_element_type=jnp.float32)
             + jnp.dot(agg, wn_ref[...], preferred_element_type=jnp.float32)
             + bs_ref[...])
        out_ref[...] = jnp.where(o >= 0, o, 0.1 * o)

    return pl.pallas_call(
        body,
        grid=(NB,),
        in_specs=[
            pl.BlockSpec((BR, H), lambda i: (i, 0)),
            pl.BlockSpec((NC, BR, H), lambda i: (0, i, 0)),
            pl.BlockSpec((NC, BR, 1), lambda i: (0, i, 0)),
            pl.BlockSpec((H, H), lambda i: (0, 0)),
            pl.BlockSpec((H, H), lambda i: (0, 0)),
            pl.BlockSpec((1, H), lambda i: (0, 0)),
        ],
        out_specs=pl.BlockSpec((BR, H), lambda i: (i, 0)),
        out_shape=jax.ShapeDtypeStruct((N, H), jnp.float32),
    )(h, aggp, degp.reshape(NC, N, 1), Ws, Wn, bs.reshape(1, H))


def _loss(hs, x, Wp, bp):
    B, H = hs.shape
    O = Wp.shape[1]

    def body(hs_ref, x_ref, wp_ref, bp_ref, out_ref):
        pred = (jnp.dot(hs_ref[...], wp_ref[...],
                        preferred_element_type=jnp.float32)
                + bp_ref[...] - x_ref[...])
        out_ref[...] = jnp.sum(pred * pred).reshape(1, 1) / (B * O)

    return pl.pallas_call(
        body,
        out_shape=jax.ShapeDtypeStruct((1, 1), jnp.float32),
    )(hs, x, Wp, bp.reshape(1, O))


def _panel_impl(nf, edge_index, tgt, x, Wm, bm, g, be,
                Ws0, Wn0, bs0, Ws1, Wn1, bs1, Wp, bp, zrows):
    src = edge_index[0]
    dst = edge_index[1]

    h = _mlp_bn(nf, Wm, bm, g, be)
    aggp, degp = _seg_sum(h, src, dst, zrows, want_deg=True)
    degp = degp.reshape(NC, h.shape[0])
    h1 = _sage_layer(h, aggp, degp, Ws0, Wn0, bs0)
    (agg1p,) = _seg_sum(h1, src, dst, zrows, want_deg=False)
    h2 = _sage_layer(h1, agg1p, degp, Ws1, Wn1, bs1)
    hs = _gather_rows(h2, tgt)
    return _loss(hs, x, Wp, bp)[0, 0]


def kernel(node_feat1, node_feat2, x1, x2, edge_index1, edge_index2,
           tgt_id1, tgt_id2,
           Wm1, bm1, g1, be1, Ws10, Wn10, bs10, Ws11, Wn11, bs11, Wp1, bp1,
           Wm2, bm2, g2, be2, Ws20, Wn20, bs20, Ws21, Wn21, bs21, Wp2, bp2):
    N, D = node_feat1.shape
    H = Wm1.shape[1]
    max_rows = N - (NS - 1) * ((N // NS) // 8 * 8)
    zrows = jnp.zeros((max_rows, H), jnp.float32)
    l1 = _panel_impl(node_feat1, edge_index1, tgt_id1, x1, Wm1, bm1, g1, be1,
                     Ws10, Wn10, bs10, Ws11, Wn11, bs11, Wp1, bp1, zrows)
    l2 = _panel_impl(node_feat2, edge_index2, tgt_id2, x2, Wm2, bm2, g2, be2,
                     Ws20, Wn20, bs20, Ws21, Wn21, bs21, Wp2, bp2, zrows)
    return jnp.stack([l1, l2])
